# baseline (speedup 1.0000x reference)
import jax
import jax.numpy as jnp
from jax import lax
from jax.experimental import pallas as pl
from jax.experimental.pallas import tpu as pltpu
from jax.experimental.pallas import tpu_sc as plsc

NC = 2
NS = 16
NW = NC * NS
CHUNK = 128
K = 4


def _gather_body(table_hbm, idx_hbm, out_hbm, idx_v, rows_v, sem):
    wid = lax.axis_index("s") * NC + lax.axis_index("c")
    rows_total = idx_hbm.shape[0]
    rows_per_w = rows_total // NW
    groups = rows_per_w // K
    base = wid * rows_per_w

    @pl.loop(0, groups)
    def _(g):
        r0 = base + g * K
        pltpu.sync_copy(idx_hbm.at[pl.ds(r0, K)], idx_v)
        copies = [
            pltpu.async_copy(table_hbm.at[idx_v.at[j]], rows_v.at[j], sem)
            for j in range(K)
        ]
        for c in copies:
            c.wait()
        pltpu.sync_copy(rows_v, out_hbm.at[pl.ds(r0, K)])


def kernel(inputs, embedding):
    batch, hist = inputs.shape
    features = embedding.shape[1]
    total = batch * hist
    nrows = total // CHUNK
    idx = inputs.reshape(nrows, CHUNK)

    gathered = pl.kernel(
        _gather_body,
        out_type=jax.ShapeDtypeStruct((nrows, CHUNK, features), jnp.float32),
        mesh=plsc.VectorSubcoreMesh(core_axis_name="c", subcore_axis_name="s"),
        scratch_types=[
            pltpu.VMEM((K, CHUNK), jnp.int32),
            pltpu.VMEM((K, CHUNK, features), jnp.float32),
            pltpu.SemaphoreType.DMA,
        ],
        compiler_params=pltpu.CompilerParams(use_tc_tiling_on_sc=False),
    )(embedding, idx)
    return gathered.reshape(batch, hist, features)


# speedup vs baseline: 1.0472x; 1.0472x over previous
import jax
import jax.numpy as jnp
from jax import lax
from jax.experimental import pallas as pl
from jax.experimental.pallas import tpu as pltpu
from jax.experimental.pallas import tpu_sc as plsc

NC = 2
NS = 16
NW = NC * NS
CHUNK = 128
K = 4


def kernel(inputs, embedding):
    batch, hist = inputs.shape
    features = embedding.shape[1]
    total = batch * hist
    assert total % (CHUNK * NW) == 0
    nrows = total // CHUNK
    rows_per_w = nrows // NW
    assert rows_per_w % K == 0
    groups = rows_per_w // K
    assert groups % 2 == 0
    idx = inputs.reshape(nrows, CHUNK)

    def body(table_hbm, idx_hbm, out_hbm, idx_v, rows0, rows1,
             sg0, sg1, so0, so1):
        rows = (rows0, rows1)
        sem_g = (sg0, sg1)
        sem_o = (so0, so1)
        wid = lax.axis_index("s") * NC + lax.axis_index("c")
        base = wid * rows_per_w
        pltpu.sync_copy(idx_hbm.at[pl.ds(base, rows_per_w)], idx_v)

        def fire_gathers(g, buf, sem):
            for j in range(K):
                pltpu.async_copy(table_hbm.at[idx_v.at[g * K + j]],
                                 buf.at[j], sem)

        def wait_gathers(g, buf, sem):
            for j in range(K):
                pltpu.make_async_copy(table_hbm.at[idx_v.at[g * K + j]],
                                      buf.at[j], sem).wait()

        def fire_out(g, buf, sem):
            pltpu.async_copy(buf, out_hbm.at[pl.ds(base + g * K, K)], sem)

        def drain_out(buf, sem):
            pltpu.make_async_copy(buf, out_hbm.at[pl.ds(base, K)], sem).wait()

        fire_gathers(0, rows[0], sem_g[0])

        @pl.loop(0, groups, step=2)
        def _(g0):
            for b in range(2):
                g = g0 + b
                nb = 1 - b

                @pl.when(g + 1 < groups)
                def _():
                    @pl.when(g >= 1)
                    def _():
                        drain_out(rows[nb], sem_o[nb])
                    fire_gathers(g + 1, rows[nb], sem_g[nb])

                wait_gathers(g, rows[b], sem_g[b])
                fire_out(g, rows[b], sem_o[b])

        drain_out(rows[0], sem_o[0])
        drain_out(rows[1], sem_o[1])

    gathered = pl.kernel(
        body,
        out_type=jax.ShapeDtypeStruct((nrows, CHUNK, features), jnp.float32),
        mesh=plsc.VectorSubcoreMesh(core_axis_name="c", subcore_axis_name="s"),
        scratch_types=[
            pltpu.VMEM((rows_per_w, CHUNK), jnp.int32),
            pltpu.VMEM((K, CHUNK, features), jnp.float32),
            pltpu.VMEM((K, CHUNK, features), jnp.float32),
            pltpu.SemaphoreType.DMA,
            pltpu.SemaphoreType.DMA,
            pltpu.SemaphoreType.DMA,
            pltpu.SemaphoreType.DMA,
        ],
        compiler_params=pltpu.CompilerParams(use_tc_tiling_on_sc=False),
    )(embedding, idx)
    return gathered.reshape(batch, hist, features)
